# baseline (speedup 1.0000x reference)
import functools

import jax
import jax.numpy as jnp
from jax import lax
from jax.experimental import pallas as pl
from jax.experimental.pallas import tpu as pltpu
from jax.experimental.pallas import tpu_sc as plsc

N_GR = 16
RN = 200
EPER = 8000
NB = 2 * N_GR
ASZ = RN * RN
REG = ASZ + 8
EPAD = 8064
K1 = RN // 2
K2 = K1 // 2
LAT = 64

_HI = lax.Precision.HIGHEST


def _dot(a, b):
    return jnp.dot(a, b, precision=_HI, preferred_element_type=jnp.float32)


def _dotl(a, b):
    return jnp.dot(a.astype(jnp.bfloat16), b.astype(jnp.bfloat16),
                   preferred_element_type=jnp.float32)


def _adj_body(idx_hbm, vals_hbm, zeros_hbm, out_hbm,
              idx_v, vals_v, buf_v, acc_sh, sem):
    c = lax.axis_index("c")
    s = lax.axis_index("s")
    b = c * N_GR + s
    base = s * REG

    pltpu.sync_copy(idx_hbm.at[b], idx_v)
    pltpu.sync_copy(vals_hbm.at[b], vals_v)

    pltpu.sync_copy(zeros_hbm, buf_v)
    pltpu.sync_copy(buf_v, acc_sh.at[pl.ds(base, ASZ)])

    pltpu.async_copy(vals_v, acc_sh.at[idx_v], sem, add=True).wait()

    pltpu.sync_copy(acc_sh.at[pl.ds(base, ASZ)], buf_v)
    pltpu.sync_copy(buf_v, out_hbm.at[b])


def _build_adj(idx, vals, zeros):
    mesh = plsc.VectorSubcoreMesh(core_axis_name="c", subcore_axis_name="s")
    f = pl.kernel(
        _adj_body,
        out_type=jax.ShapeDtypeStruct((NB, ASZ), jnp.float32),
        mesh=mesh,
        scratch_types=[
            pltpu.VMEM((EPAD,), jnp.int32),
            pltpu.VMEM((EPAD,), jnp.float32),
            pltpu.VMEM((ASZ,), jnp.float32),
            pltpu.VMEM_SHARED((N_GR * REG,), jnp.float32),
            pltpu.SemaphoreType.DMA,
        ],
    )
    return f(idx, vals, zeros)


def _col2row(v):
    return jnp.swapaxes(v, 0, 1)


def _cheb(A_norm, x, Wf, b_row, lmul_dot):
    f = Wf.shape[0] // 3
    tx1 = -lmul_dot(A_norm, x)
    tx2 = -2.0 * lmul_dot(A_norm, tx1) - x
    return (_dotl(x, Wf[0:f]) + _dotl(tx1, Wf[f:2 * f])
            + _dotl(tx2, Wf[2 * f:3 * f]) + b_row)


def _norm_adj(A, n):
    deg = jnp.sum(A, axis=1, keepdims=True)
    pos = deg > 0.0
    dinv = jnp.where(pos, lax.rsqrt(jnp.where(pos, deg, 1.0)), 0.0)
    return A * dinv * _col2row(dinv)


def _select_mask(score, n, k):
    i = lax.broadcasted_iota(jnp.int32, (n, n), 0)
    j = lax.broadcasted_iota(jnp.int32, (n, n), 1)
    s_row = _col2row(score)
    beats = (s_row > score) | ((s_row == score) & (j < i))
    rank = jnp.sum(beats.astype(jnp.float32), axis=1, keepdims=True)
    return (rank < float(k)).astype(jnp.float32)


def _select_matrices(keptf, n, k):
    i = lax.broadcasted_iota(jnp.int32, (n, n), 0)
    j = lax.broadcasted_iota(jnp.int32, (n, n), 1)
    lstrict = (j < i).astype(jnp.float32)
    posn = _dot(lstrict, keptf)
    kept = keptf > 0.5
    pk = lax.broadcasted_iota(jnp.int32, (k, 1), 0).astype(jnp.float32)
    S = ((pk == _col2row(posn)) & (_col2row(keptf) > 0.5)).astype(jnp.float32)
    rk = lax.broadcasted_iota(jnp.int32, (1, k), 1).astype(jnp.float32)
    St = ((posn == rk) & kept).astype(jnp.float32)
    return S, St


def _sigmoid(z):
    return 1.0 / (1.0 + jnp.exp(-z))


def _score(x, pw_ref):
    pw = pw_ref[...]
    wnorm = jnp.sqrt(jnp.sum(pw * pw))
    pw_col = jnp.swapaxes(pw, 0, 1)
    return _sigmoid(_dotl(x, pw_col) / wnorm)


GPB = 4

_NEG = -3.0e38


def _graph_body(a_ref, x_ref, w1_ref, b1_ref, pw1_ref, w2_ref, b2_ref, pw2_ref,
                h_ref):
    ikk = lax.broadcasted_iota(jnp.int32, (K1, K1), 0)
    jkk = lax.broadcasted_iota(jnp.int32, (K1, K1), 1)
    eye_k = (ikk == jkk).astype(jnp.float32)
    for t in range(GPB):
        A = a_ref[t]
        x = x_ref[t]

        xc = _cheb(_norm_adj(A, RN), x, w1_ref[...], b1_ref[...], _dot)

        s1 = _score(xc, pw1_ref)
        k1 = _select_mask(s1, RN, K1)
        xs = xc * s1

        h_ref[t, 0:1, 0:64] = jnp.max(
            jnp.where(k1 > 0.5, xs, _NEG), axis=0, keepdims=True)
        h_ref[t, 0:1, 64:128] = jnp.sum(xs * k1, axis=0,
                                        keepdims=True) / float(K1)

        S1, St1 = _select_matrices(k1, RN, K1)
        xp = _dot(S1, xs)
        Ap = _dot(S1, _dot(A, St1))

        aaug = Ap + eye_k
        A2 = _dotl(aaug, aaug) * (1.0 - eye_k)

        xc2 = _cheb(_norm_adj(A2, K1), xp, w2_ref[...], b2_ref[...], _dotl)

        s2 = _score(xc2, pw2_ref)
        k2 = _select_mask(s2, K1, K2)
        xs2 = xc2 * s2

        h_ref[t, 0:1, 128:192] = jnp.max(
            jnp.where(k2 > 0.5, xs2, _NEG), axis=0, keepdims=True)
        h_ref[t, 0:1, 192:256] = jnp.sum(xs2 * k2, axis=0,
                                         keepdims=True) / float(K2)


def _graph_tc(A_all, X_all, w1, b1, pw1, w2, b2, pw2):
    return pl.pallas_call(
        _graph_body,
        grid=(NB // GPB,),
        in_specs=[
            pl.BlockSpec((GPB, RN, RN), lambda g: (g, 0, 0)),
            pl.BlockSpec((GPB, RN, RN), lambda g: (g, 0, 0)),
            pl.BlockSpec((3 * RN, LAT), lambda g: (0, 0)),
            pl.BlockSpec((1, LAT), lambda g: (0, 0)),
            pl.BlockSpec((1, LAT), lambda g: (0, 0)),
            pl.BlockSpec((3 * LAT, LAT), lambda g: (0, 0)),
            pl.BlockSpec((1, LAT), lambda g: (0, 0)),
            pl.BlockSpec((1, LAT), lambda g: (0, 0)),
        ],
        out_specs=pl.BlockSpec((GPB, 1, 256), lambda g: (g, 0, 0)),
        out_shape=jax.ShapeDtypeStruct((NB, 1, 256), jnp.float32),
    )(A_all, X_all, w1, b1, pw1, w2, b2, pw2).reshape(NB, 256)


def _head_body(h_ref, fcw_ref, fcb_ref, bng_ref, bnb_ref, c1_ref, c2_ref,
               c2b_ref, o1_ref, o2_ref, f1_ref, f2_ref):
    h = jax.nn.relu(_dotl(h_ref[...], fcw_ref[...]) + fcb_ref[...])
    for v, (f_ref, o_ref) in enumerate(((f1_ref, o1_ref), (f2_ref, o2_ref))):
        hv = h[v * N_GR:(v + 1) * N_GR]
        mu = jnp.sum(hv, axis=0, keepdims=True) / float(N_GR)
        d = hv - mu
        var = jnp.sum(d * d, axis=0, keepdims=True) / float(N_GR)
        hn = d * lax.rsqrt(var + 1e-5) * bng_ref[...] + bnb_ref[...]
        out = _dotl(jax.nn.relu(_dotl(hn, c1_ref[...])), c2_ref[...]) + c2b_ref[...]
        fn = jnp.maximum(jnp.sqrt(jnp.sum(hn * hn, axis=1, keepdims=True)), 1e-12)
        on = jnp.maximum(jnp.sqrt(jnp.sum(out * out, axis=1, keepdims=True)), 1e-12)
        f_ref[...] = hn / fn
        o_ref[...] = out / on


def _head_tc(H, fcw, fcb, bng, bnb, c1w, c2w, c2b):
    return pl.pallas_call(
        _head_body,
        out_shape=(
            jax.ShapeDtypeStruct((N_GR, 512), jnp.float32),
            jax.ShapeDtypeStruct((N_GR, 512), jnp.float32),
            jax.ShapeDtypeStruct((N_GR, 256), jnp.float32),
            jax.ShapeDtypeStruct((N_GR, 256), jnp.float32),
        ),
    )(H, fcw, fcb, bng, bnb, c1w, c2w, c2b)


def _edge_blocks(edge_index, edge_attr):
    src = edge_index[0].reshape(N_GR, EPER)
    dst = edge_index[1].reshape(N_GR, EPER)
    g = jnp.arange(N_GR, dtype=jnp.int32)[:, None]
    flat = RN * src + dst - (RN * RN + RN) * g + g * REG
    flat = jnp.pad(flat, ((0, 0), (0, EPAD - EPER)),
                   constant_values=ASZ)
    vals = jnp.pad(edge_attr.reshape(N_GR, EPER), ((0, 0), (0, EPAD - EPER)))
    return flat.astype(jnp.int32), vals


def kernel(x1, edge_index1, edge_attr1, batch1,
           x2, edge_index2, edge_attr2, batch2, params):
    i1, v1 = _edge_blocks(edge_index1, edge_attr1)
    i2, v2 = _edge_blocks(edge_index2, edge_attr2)
    idx = jnp.concatenate([i1, i2], axis=0)
    vals = jnp.concatenate([v1, v2], axis=0)
    zeros = jnp.zeros((ASZ,), jnp.float32)

    A_all = _build_adj(idx, vals, zeros).reshape(NB, RN, RN)

    X_all = jnp.concatenate([x1.reshape(N_GR, RN, RN),
                             x2.reshape(N_GR, RN, RN)], axis=0)

    p = params
    H = _graph_tc(A_all, X_all,
                  p['conv1_W'].reshape(3 * RN, LAT),
                  p['conv1_b'].reshape(1, LAT),
                  p['pool1_w'].reshape(1, LAT),
                  p['conv2_W'].reshape(3 * LAT, LAT),
                  p['conv2_b'].reshape(1, LAT),
                  p['pool2_w'].reshape(1, LAT))

    o1, o2, f1, f2 = _head_tc(H, p['fc_W'], p['fc_b'].reshape(1, 256),
                              p['bn_g'].reshape(1, 256),
                              p['bn_b'].reshape(1, 256),
                              p['c1_W'], p['c2_W'], p['c2_b'].reshape(1, 512))
    return (o1, o2, f1, f2)
